# baseline (speedup 1.0000x reference)
import jax
import jax.numpy as jnp
from jax import lax
from jax.experimental import pallas as pl
from jax.experimental.pallas import tpu as pltpu
from jax.experimental.pallas import tpu_sc as plsc

_L = 16
_CA = 16
_D = 512
_R = 3 * _D
_CW = _CA * _R
_WIN = (_L - 1) * _R + 1


def _sc_body(x_hbm, der_hbm, out_hbm, x_v, der_v0, der_v1, red_v, out_v,
             sem0, sem1):
    n_chunks = 8
    batch_w = n_chunks * _CW
    wid = lax.axis_index("c") * 16 + lax.axis_index("s")
    lane = lax.broadcasted_iota(jnp.int32, (_L,), 0)
    lrow = lane * _R

    pltpu.sync_copy(x_hbm.at[pl.ds(wid * (2 * _D), 2 * _D)], x_v)

    bufs = (der_v0, der_v1)
    sems = (sem0, sem1)
    zero = jnp.zeros((_L,), jnp.float32)
    out_vec = zero

    for bl in range(2):
        base = (wid * 2 + bl) * batch_w
        for u in range(2):
            pltpu.async_copy(der_hbm.at[pl.ds(base + u * _CW, _CW)], bufs[u],
                             sems[u])

        def pair_body(t, sq_acc, bl=bl, base=base):
            for u in range(2):
                c = 2 * t + u
                buf, sem = bufs[u], sems[u]
                pltpu.make_async_copy(
                    der_hbm.at[pl.ds(base, _CW)], buf, sem).wait()

                def dloop(i, carry, bl=bl, buf=buf):
                    a0, a1, a2, b0, b1, b2, i0, i1, i2 = carry
                    xv = x_v[pl.ds(bl * _D + i * _L, _L)]
                    for m in range(_L):
                        g0 = plsc.load_gather(buf, [i0])
                        g1 = plsc.load_gather(buf, [i1])
                        g2 = plsc.load_gather(buf, [i2])
                        i0, i1, i2 = i0 + 3, i1 + 3, i2 + 3
                        xs = xv[m]
                        if m % 2 == 0:
                            a0 = a0 + g0 * xs
                            a1 = a1 + g1 * xs
                            a2 = a2 + g2 * xs
                        else:
                            b0 = b0 + g0 * xs
                            b1 = b1 + g1 * xs
                            b2 = b2 + g2 * xs
                    return (a0, a1, a2, b0, b1, b2, i0, i1, i2)

                a0, a1, a2, b0, b1, b2, _, _, _ = lax.fori_loop(
                    0, _D // _L, dloop,
                    (zero,) * 6 + (lrow, lrow + 1, lrow + 2))
                a0, a1, a2 = a0 + b0, a1 + b1, a2 + b2
                sq_acc = sq_acc + a0 * a0 + a1 * a1 + a2 * a2

                @pl.when(c + 2 < n_chunks)
                def _():
                    pltpu.async_copy(
                        der_hbm.at[pl.ds(base + (c + 2) * _CW, _CW)], buf,
                        sem)

            return sq_acc

        sq_acc = lax.fori_loop(0, n_chunks // 2, pair_body, zero)

        for s in (8, 4, 2, 1):
            red_v[...] = sq_acc
            sq_acc = sq_acc + plsc.load_gather(red_v, [(lane + s) % _L])
        out_vec = jnp.where(lane == bl, sq_acc, out_vec)

    out_v[...] = out_vec
    pltpu.sync_copy(out_v, out_hbm.at[pl.ds(wid * _L, _L)])


def kernel(x, der_desc_wrt_coord):
    B, A, D, K = der_desc_wrt_coord.shape
    der_flat = der_desc_wrt_coord.reshape(B * A * D * K)
    x_flat = x.reshape(B * D)
    f = pl.kernel(
        _sc_body,
        out_type=jax.ShapeDtypeStruct((32 * _L,), jnp.float32),
        mesh=plsc.VectorSubcoreMesh(core_axis_name="c", subcore_axis_name="s"),
        compiler_params=pltpu.CompilerParams(needs_layout_passes=False),
        scratch_types=[
            pltpu.VMEM((2 * _D,), jnp.float32),
            pltpu.VMEM((_CW,), jnp.float32),
            pltpu.VMEM((_CW,), jnp.float32),
            pltpu.VMEM((_L,), jnp.float32),
            pltpu.VMEM((_L,), jnp.float32),
            pltpu.SemaphoreType.DMA,
            pltpu.SemaphoreType.DMA,
        ],
    )
    out2 = f(x_flat, der_flat)
    return out2.reshape(32, _L)[:, :2].reshape(B)


# speedup vs baseline: 12.6759x; 12.6759x over previous
import jax
import jax.numpy as jnp
from jax import lax
from jax.experimental import pallas as pl
from jax.experimental.pallas import tpu as pltpu
from jax.experimental.pallas import tpu_sc as plsc

_L = 16
_CA = 16
_D = 512
_R = 3 * _D
_CW = _CA * _R
_WIN = (_L - 1) * _R + 1


def _sc_body(x_hbm, der_hbm, out_hbm, x_v, der_v0, der_v1, red_v, out_v,
             sem0, sem1):
    n_chunks = 128 // _CA
    wid = lax.axis_index("c") * 16 + lax.axis_index("s")
    lane = lax.broadcasted_iota(jnp.int32, (_L,), 0)
    lrow = lane * _R

    pltpu.sync_copy(x_hbm.at[pl.ds(wid * (2 * _D), 2 * _D)], x_v)

    bufs = (der_v0, der_v1)
    sems = (sem0, sem1)
    zero = jnp.zeros((_L,), jnp.float32)
    out_vec = zero

    for bl in range(2):
        b = wid * 2 + bl
        for u in range(2):
            pltpu.async_copy(der_hbm.at[b, pl.ds(u * _CW, _CW)], bufs[u],
                             sems[u])

        def pair_body(t, sq_acc, bl=bl, b=b):
            for u in range(2):
                c = 2 * t + u
                buf, sem = bufs[u], sems[u]
                pltpu.make_async_copy(
                    der_hbm.at[b, pl.ds(0, _CW)], buf, sem).wait()

                def dloop(i, carry, bl=bl, buf=buf):
                    a0, a1, a2, b0, b1, b2, i0, i1, i2 = carry
                    xv = x_v[pl.ds(bl * _D + i * _L, _L)]
                    for m in range(_L):
                        g0 = plsc.load_gather(buf, [i0])
                        g1 = plsc.load_gather(buf, [i1])
                        g2 = plsc.load_gather(buf, [i2])
                        i0, i1, i2 = i0 + 3, i1 + 3, i2 + 3
                        xs = xv[m]
                        if m % 2 == 0:
                            a0 = a0 + g0 * xs
                            a1 = a1 + g1 * xs
                            a2 = a2 + g2 * xs
                        else:
                            b0 = b0 + g0 * xs
                            b1 = b1 + g1 * xs
                            b2 = b2 + g2 * xs
                    return (a0, a1, a2, b0, b1, b2, i0, i1, i2)

                a0, a1, a2, b0, b1, b2, _, _, _ = lax.fori_loop(
                    0, _D // _L, dloop,
                    (zero,) * 6 + (lrow, lrow + 1, lrow + 2))
                a0, a1, a2 = a0 + b0, a1 + b1, a2 + b2
                sq_acc = sq_acc + a0 * a0 + a1 * a1 + a2 * a2

                @pl.when(c + 2 < n_chunks)
                def _():
                    pltpu.async_copy(
                        der_hbm.at[b, pl.ds((c + 2) * _CW, _CW)], buf, sem)

            return sq_acc

        sq_acc = lax.fori_loop(0, n_chunks // 2, pair_body, zero)

        for s in (8, 4, 2, 1):
            red_v[...] = sq_acc
            sq_acc = sq_acc + plsc.load_gather(red_v, [(lane + s) % _L])
        out_vec = jnp.where(lane == bl, sq_acc, out_vec)

    out_v[...] = out_vec
    pltpu.sync_copy(out_v, out_hbm.at[pl.ds(wid * _L, _L)])


def kernel(x, der_desc_wrt_coord):
    B, A, D, K = der_desc_wrt_coord.shape
    der2 = der_desc_wrt_coord.reshape(B, A * D * K)
    x_flat = x.reshape(B * D)
    f = pl.kernel(
        _sc_body,
        out_type=jax.ShapeDtypeStruct((32 * _L,), jnp.float32),
        mesh=plsc.VectorSubcoreMesh(core_axis_name="c", subcore_axis_name="s"),
        compiler_params=pltpu.CompilerParams(needs_layout_passes=False),
        scratch_types=[
            pltpu.VMEM((2 * _D,), jnp.float32),
            pltpu.VMEM((_CW,), jnp.float32),
            pltpu.VMEM((_CW,), jnp.float32),
            pltpu.VMEM((_L,), jnp.float32),
            pltpu.VMEM((_L,), jnp.float32),
            pltpu.SemaphoreType.DMA,
            pltpu.SemaphoreType.DMA,
        ],
    )
    out2 = f(x_flat, der2)
    return out2.reshape(32, _L)[:, :2].reshape(B)


# speedup vs baseline: 37.1578x; 2.9314x over previous
import jax
import jax.numpy as jnp
from jax import lax
from jax.experimental import pallas as pl


def _body(w3_ref, der_ref, out_ref):
    y = lax.dot_general(der_ref[0], w3_ref[0], (((1,), (1,)), ((), ())),
                        preferred_element_type=jnp.float32)
    out_ref[...] = jnp.sum(y * y, keepdims=True)[None]


def kernel(x, der_desc_wrt_coord):
    B, A, D, K = der_desc_wrt_coord.shape
    der2 = der_desc_wrt_coord.reshape(B, A, D * K)
    j = jnp.arange(D * K, dtype=jnp.int32)
    sel = (j[None, :] % K) == jnp.arange(K, dtype=jnp.int32)[:, None]
    w3 = jnp.where(sel[None], x[:, None, j // K], 0.0)
    out = pl.pallas_call(
        _body,
        grid=(B,),
        in_specs=[
            pl.BlockSpec((1, K, D * K), lambda b: (b, 0, 0)),
            pl.BlockSpec((1, A, D * K), lambda b: (b, 0, 0)),
        ],
        out_specs=pl.BlockSpec((1, 1, 1), lambda b: (b, 0, 0)),
        out_shape=jax.ShapeDtypeStruct((B, 1, 1), jnp.float32),
    )(w3, der2)
    return out[:, 0, 0]
